# baseline (speedup 1.0000x reference)
import functools

import jax
import jax.numpy as jnp
from jax import lax
from jax.experimental import pallas as pl
from jax.experimental.pallas import tpu as pltpu
from jax.experimental.pallas import tpu_sc as plsc

NC = 2
NS = 16
NW = NC * NS
C = 80


def _deg_body(nchunk, npad, dst3, ew3, out, dst_v, ew_v, zbuf, acc):
  cid = lax.axis_index("c")
  sid = lax.axis_index("s")
  t = cid * NS + sid
  per = npad // NS
  pltpu.sync_copy(dst3.at[t], dst_v)
  pltpu.sync_copy(ew3.at[t], ew_v)
  for i in range(per // 16):
    zbuf[pl.ds(i * 16, 16)] = jnp.zeros((16,), jnp.float32)
  pltpu.sync_copy(zbuf, acc.at[pl.ds(sid * per, per)])
  plsc.subcore_barrier()

  def chunk(ci, carry):
    pltpu.sync_copy(ew_v.at[ci], acc.at[dst_v.at[ci]], add=True)
    return carry

  lax.fori_loop(0, nchunk, chunk, 0)
  plsc.subcore_barrier()
  pltpu.sync_copy(acc.at[pl.ds(sid * per, per)],
                  out.at[cid, pl.ds(sid * per, per)])


def _make_agg(nchunk, n, npad, compute_norm, packed=False):
  K = nchunk // 4

  def body(*refs):
    if compute_norm:
      (src3, dst3, ew3, degp3, xw, norm3, part,
       src_c, dst_c, aux_c, dinv_v, rows, normb,
       g_sem, s_sem, i_sem, n_sem, acc) = refs
      aux3 = ew3
    elif packed:
      (src3, dst3, norm3, xw, part,
       src_c, dst_c, aux_c, rows_bf, rows,
       g_sem, s_sem, i_sem, acc) = refs
      aux3 = norm3
    else:
      (src3, dst3, norm3, xw, part,
       src_c, dst_c, aux_c, rows,
       g_sem, s_sem, i_sem, acc) = refs
      aux3 = norm3
    cid = lax.axis_index("c")
    sid = lax.axis_index("s")
    t = cid * NS + sid
    rows_per = npad // NS
    d = rows.shape[2]

    if compute_norm:
      pltpu.sync_copy(degp3.at[0], rows.at[0])
      pltpu.sync_copy(degp3.at[1], rows.at[1])

      def dfill(r, carry):
        for jj in range(d // 16):
          slc = pl.ds(jj * 16, 16)
          deg = rows[0, r, slc] + rows[1, r, slc]
          x = jnp.maximum(deg, 1e-12)
          yi = 0x5F3759DF - (plsc.bitcast(x, jnp.int32) >> 1)
          y = plsc.bitcast(yi, jnp.float32)
          hx = 0.5 * x
          y = y * (1.5 - hx * y * y)
          y = y * (1.5 - hx * y * y)
          y = y * (1.5 - hx * y * y)
          y = jnp.where(deg > 0.0, y, 0.0)
          dinv_v[pl.ds(r * d + jj * 16, 16)] = y
        return carry

      lax.fori_loop(0, rows.shape[1], dfill, 0)

    def zfill(r, carry):
      for jj in range(d // 16):
        rows[0, r, pl.ds(jj * 16, 16)] = jnp.zeros((16,), jnp.float32)
      return carry

    lax.fori_loop(0, C, zfill, 0)
    for k in range(rows_per // C):
      pltpu.sync_copy(rows.at[0], acc.at[pl.ds(sid * rows_per + k * C, C)])
    plsc.subcore_barrier()

    def idx_start(cn, slot):
      pltpu.async_copy(src3.at[t, cn], src_c.at[slot], i_sem.at[slot])
      pltpu.async_copy(dst3.at[t, cn], dst_c.at[slot], i_sem.at[slot])
      pltpu.async_copy(aux3.at[t, cn], aux_c.at[slot], i_sem.at[slot])

    def idx_wait(cn, slot):
      pltpu.make_async_copy(src3.at[t, cn], src_c.at[slot],
                            i_sem.at[slot]).wait()
      pltpu.make_async_copy(dst3.at[t, cn], dst_c.at[slot],
                            i_sem.at[slot]).wait()
      pltpu.make_async_copy(aux3.at[t, cn], aux_c.at[slot],
                            i_sem.at[slot]).wait()

    grows = rows_bf if packed else rows

    def gather_start(slot, p):
      pltpu.async_copy(xw.at[src_c.at[slot]], grows.at[p], g_sem.at[p])

    def gather_wait(p):
      pltpu.make_async_copy(xw.at[pl.ds(0, C)], grows.at[p],
                            g_sem.at[p]).wait()

    def scatter_start(slot, p):
      pltpu.async_copy(rows.at[p], acc.at[dst_c.at[slot]], s_sem.at[p],
                       priority=1, add=True)

    def scatter_wait(p):
      pltpu.make_async_copy(rows.at[p], acc.at[pl.ds(0, C)],
                            s_sem.at[p]).wait()

    pltpu.sync_copy(src3.at[t, 0], src_c.at[0])
    pltpu.sync_copy(dst3.at[t, 0], dst_c.at[0])
    pltpu.sync_copy(aux3.at[t, 0], aux_c.at[0])
    gather_start(0, 0)
    idx_start(1, 1)

    def quad(kk, carry):
      for j in range(4):
        ci = kk * 4 + j
        p = j % 2
        q = (j + 1) % 2
        gather_wait(p)
        if not packed:
          if j == 0:
            @pl.when(kk > 0)
            def _():
              scatter_wait(q)
          else:
            scatter_wait(q)
        def issue_next():
          idx_wait(ci + 1, (j + 1) % 4)
          gather_start((j + 1) % 4, q)

        def issue_idx2():
          idx_start(ci + 2, (j + 2) % 4)

        if packed:
          if j < 3:
            issue_next()
          else:
            @pl.when(kk < K - 1)
            def _():
              issue_next()
          if j < 2:
            @pl.when(kk > 0)
            def _():
              scatter_wait(p)
          else:
            scatter_wait(p)
          if j < 2:
            issue_idx2()
          else:
            @pl.when(kk < K - 1)
            def _():
              issue_idx2()
        else:
          if j < 3:
            issue_next()
            if j < 2:
              issue_idx2()
            else:
              @pl.when(kk < K - 1)
              def _():
                issue_idx2()
          else:
            @pl.when(kk < K - 1)
            def _():
              issue_next()
              issue_idx2()

        if compute_norm:
          if j < 2:
            @pl.when(kk > 0)
            def _():
              pltpu.make_async_copy(normb.at[p], norm3.at[t, ci],
                                    n_sem.at[p]).wait()
          else:
            pltpu.make_async_copy(normb.at[p], norm3.at[t, ci],
                                  n_sem.at[p]).wait()

          def ngroup(g, c2):
            s16 = src_c[j, pl.ds(g * 16, 16)]
            d16 = dst_c[j, pl.ds(g * 16, 16)]
            e16 = aux_c[j, pl.ds(g * 16, 16)]
            normb[p, pl.ds(g * 16, 16)] = (
                plsc.load_gather(dinv_v, [s16]) * e16 *
                plsc.load_gather(dinv_v, [d16]))
            return c2

          lax.fori_loop(0, C // 16, ngroup, 0)
          pltpu.async_copy(normb.at[p], norm3.at[t, ci], n_sem.at[p])

        if packed:
          iota16 = lax.iota(jnp.int32, 16)

          @plsc.parallel_loop(0, C, unroll=4)
          def scale(ee):
            nsp = plsc.load_gather(
                aux_c, [jnp.full((16,), j, jnp.int32),
                        jnp.full((16,), ee, jnp.int32)])
            fp = jnp.full((16,), p, jnp.int32)
            fe = jnp.full((16,), ee, jnp.int32)
            for jj in range(d // 32):
              w32 = rows_bf[p, ee, pl.ds(jj * 32, 32)]
              w16 = plsc.bitcast(w32, jnp.int32)
              ev = plsc.bitcast(w16 << 16, jnp.float32) * nsp
              od = plsc.bitcast(w16 & jnp.int32(-65536), jnp.float32) * nsp
              plsc.store_scatter(
                  rows, [fp, fe, iota16 * 2 + (jj * 32)], ev)
              plsc.store_scatter(
                  rows, [fp, fe, iota16 * 2 + (jj * 32 + 1)], od)
        else:

          @plsc.parallel_loop(0, C, unroll=8)
          def scale(ee):
            if compute_norm:
              nsp = plsc.load_gather(
                  normb, [jnp.full((16,), p, jnp.int32),
                          jnp.full((16,), ee, jnp.int32)])
            else:
              nsp = plsc.load_gather(
                  aux_c, [jnp.full((16,), j, jnp.int32),
                          jnp.full((16,), ee, jnp.int32)])
            for jj in range(d // 16):
              slc = pl.ds(jj * 16, 16)
              rows[p, ee, slc] = rows[p, ee, slc] * nsp

        scatter_start(j, p)
      return carry

    lax.fori_loop(0, K, quad, 0)
    if packed:
      scatter_wait(0)
    scatter_wait(1)
    if compute_norm:
      pltpu.make_async_copy(normb.at[0], norm3.at[t, nchunk - 2],
                            n_sem.at[0]).wait()
      pltpu.make_async_copy(normb.at[1], norm3.at[t, nchunk - 1],
                            n_sem.at[1]).wait()
    plsc.subcore_barrier()
    pltpu.sync_copy(acc.at[pl.ds(sid * rows_per, rows_per)],
                    part.at[cid, pl.ds(sid * rows_per, rows_per)])

  return body


def kernel(x, edge_index, edge_weight, train_mask, y, W1, b1, W2, b2):
  n, d = x.shape
  e = edge_weight.shape[0]
  npad = ((n + 16 * 128 - 1) // (16 * 128)) * (16 * 128)
  ef = e + n
  epad = ((ef + NW * C * 4 - 1) // (NW * C * 4)) * (NW * C * 4)
  nchunk = epad // (NW * C)
  f32 = jnp.float32

  loop = jnp.arange(n, dtype=jnp.int32)
  padn = epad - ef
  padidx = jnp.arange(padn, dtype=jnp.int32) % n
  src3 = jnp.concatenate([edge_index[0], loop, padidx]).reshape(NW, nchunk, C)
  dst3 = jnp.concatenate([edge_index[1], loop, padidx]).reshape(NW, nchunk, C)
  ew3 = jnp.concatenate(
      [edge_weight, jnp.ones((n,), f32), jnp.zeros((padn,), f32)]
  ).reshape(NW, nchunk, C)
  xpad = jnp.pad(x, ((0, npad - n), (0, 0)))

  mesh = plsc.VectorSubcoreMesh(
      core_axis_name="c", subcore_axis_name="s", num_cores=NC,
      num_subcores=NS)

  degp = pl.kernel(
      functools.partial(_deg_body, nchunk, npad),
      out_type=jax.ShapeDtypeStruct((NC, npad), f32),
      mesh=mesh,
      compiler_params=pltpu.CompilerParams(needs_layout_passes=False),
      scratch_types=[
          pltpu.VMEM((nchunk, C), jnp.int32),
          pltpu.VMEM((nchunk, C), f32),
          pltpu.VMEM((npad // NS,), f32),
          pltpu.VMEM_SHARED((npad,), f32),
      ],
  )(dst3, ew3)
  degp3 = degp.reshape(NC, npad // 128, 128)

  nb = npad // 1024

  def mm1_body(x_ref, w_ref, xw_ref):
    xw_ref[...] = jnp.dot(x_ref[...], w_ref[...],
                          preferred_element_type=f32)

  xw1 = pl.pallas_call(
      mm1_body,
      grid=(nb,),
      in_specs=[
          pl.BlockSpec((1024, d), lambda i: (i, 0)),
          pl.BlockSpec((d, d), lambda i: (0, 0)),
      ],
      out_specs=pl.BlockSpec((1024, d), lambda i: (i, 0)),
      out_shape=jax.ShapeDtypeStruct((npad, d), f32),
  )(xpad, W1)

  agg1 = pl.kernel(
      _make_agg(nchunk, n, npad, True),
      out_type=[
          jax.ShapeDtypeStruct((NW, nchunk, C), f32),
          jax.ShapeDtypeStruct((NC, npad, d), f32),
      ],
      mesh=mesh,
      compiler_params=pltpu.CompilerParams(needs_layout_passes=False),
      scratch_types=[
          pltpu.VMEM((4, C), jnp.int32),
          pltpu.VMEM((4, C), jnp.int32),
          pltpu.VMEM((4, C), f32),
          pltpu.VMEM((npad,), f32),
          pltpu.VMEM((2, C, d), f32),
          pltpu.VMEM((2, C), f32),
          pltpu.SemaphoreType.DMA((2,)),
          pltpu.SemaphoreType.DMA((2,)),
          pltpu.SemaphoreType.DMA((4,)),
          pltpu.SemaphoreType.DMA((2,)),
          pltpu.VMEM_SHARED((npad, d), f32),
      ],
  )
  norm3, part1 = agg1(src3, dst3, ew3, degp3, xw1)

  gb = 512
  ng = npad // gb

  def mm2_body(p_ref, b_ref, w_ref, out_ref):
    h = jnp.maximum(p_ref[0] + p_ref[1] + b_ref[...], 0.0)
    out_ref[...] = jnp.dot(h, w_ref[...], preferred_element_type=f32)

  xw2 = pl.pallas_call(
      mm2_body,
      grid=(ng,),
      in_specs=[
          pl.BlockSpec((NC, gb, d), lambda i: (0, i, 0)),
          pl.BlockSpec((1, d), lambda i: (0, 0)),
          pl.BlockSpec((d, d), lambda i: (0, 0)),
      ],
      out_specs=pl.BlockSpec((gb, d), lambda i: (i, 0)),
      out_shape=jax.ShapeDtypeStruct((npad, d), f32),
  )(part1, b1.reshape(1, d), W2)
  agg2 = pl.kernel(
      _make_agg(nchunk, n, npad, False),
      out_type=jax.ShapeDtypeStruct((NC, npad, d), f32),
      mesh=mesh,
      compiler_params=pltpu.CompilerParams(needs_layout_passes=False),
      scratch_types=[
          pltpu.VMEM((4, C), jnp.int32),
          pltpu.VMEM((4, C), jnp.int32),
          pltpu.VMEM((4, C), f32),
          pltpu.VMEM((2, C, d), f32),
          pltpu.SemaphoreType.DMA((2,)),
          pltpu.SemaphoreType.DMA((2,)),
          pltpu.SemaphoreType.DMA((4,)),
          pltpu.VMEM_SHARED((npad, d), f32),
      ],
  )
  part2 = agg2(src3, dst3, norm3, xw2)

  maskp = jnp.pad(train_mask, (0, npad - n))
  maskb = jnp.broadcast_to(maskp[:, None], (npad, d))

  def out_body(p_ref, b_ref, m_ref, out_ref):
    h = p_ref[0] + p_ref[1] + b_ref[...]
    out_ref[...] = jnp.where(m_ref[...], h, 0.0)

  h_sel = pl.pallas_call(
      out_body,
      grid=(ng,),
      in_specs=[
          pl.BlockSpec((NC, gb, d), lambda i: (0, i, 0)),
          pl.BlockSpec((1, d), lambda i: (0, 0)),
          pl.BlockSpec((gb, d), lambda i: (i, 0)),
      ],
      out_specs=pl.BlockSpec((gb, d), lambda i: (i, 0)),
      out_shape=jax.ShapeDtypeStruct((npad, d), f32),
  )(part2, b2.reshape(1, d), maskb)

  y_sel = jnp.where(train_mask, y, jnp.zeros((), dtype=y.dtype))
  return (h_sel[:n], y_sel)


# speedup vs baseline: 1.0017x; 1.0017x over previous
import functools

import jax
import jax.numpy as jnp
from jax import lax
from jax.experimental import pallas as pl
from jax.experimental.pallas import tpu as pltpu
from jax.experimental.pallas import tpu_sc as plsc

NC = 2
NS = 16
NW = NC * NS
C = 80


def _deg_body(nchunk, npad, dst3, ew3, out, dst_v, ew_v, zbuf, w_sem, acc):
  cid = lax.axis_index("c")
  sid = lax.axis_index("s")
  t = cid * NS + sid
  per = npad // NS
  pltpu.sync_copy(dst3.at[t], dst_v)
  pltpu.sync_copy(ew3.at[t], ew_v)
  for i in range(per // 16):
    zbuf[pl.ds(i * 16, 16)] = jnp.zeros((16,), jnp.float32)
  pltpu.sync_copy(zbuf, acc.at[pl.ds(sid * per, per)])
  plsc.subcore_barrier()

  def wstart(w, s):
    for b in range(4):
      pltpu.async_copy(ew_v.at[w * 4 + b], acc.at[dst_v.at[w * 4 + b]],
                       w_sem.at[s], add=True)

  def wdrain(w, s):
    for b in range(4):
      pltpu.make_async_copy(ew_v.at[w * 4 + b], acc.at[pl.ds(0, C)],
                            w_sem.at[s]).wait()

  nw = nchunk // 4
  wstart(0, 0)

  def wave(w, carry):
    @pl.when(w < nw - 1)
    def _():
      wstart(w + 1, (w + 1) % 2)
    wdrain(w, w % 2)
    return carry

  lax.fori_loop(0, nw, wave, 0)
  plsc.subcore_barrier()
  pltpu.sync_copy(acc.at[pl.ds(sid * per, per)],
                  out.at[cid, pl.ds(sid * per, per)])


def _make_agg(nchunk, n, npad, compute_norm, packed=False):
  K = nchunk // 4

  def body(*refs):
    if compute_norm:
      (src3, dst3, ew3, degp3, xw, norm3, part,
       src_c, dst_c, aux_c, dinv_v, rows, normb,
       g_sem, s_sem, i_sem, n_sem, acc) = refs
      aux3 = ew3
    elif packed:
      (src3, dst3, norm3, xw, part,
       src_c, dst_c, aux_c, rows_bf, rows,
       g_sem, s_sem, i_sem, acc) = refs
      aux3 = norm3
    else:
      (src3, dst3, norm3, xw, part,
       src_c, dst_c, aux_c, rows,
       g_sem, s_sem, i_sem, acc) = refs
      aux3 = norm3
    cid = lax.axis_index("c")
    sid = lax.axis_index("s")
    t = cid * NS + sid
    rows_per = npad // NS
    d = rows.shape[2]

    if compute_norm:
      pltpu.sync_copy(degp3.at[0], rows.at[0])
      pltpu.sync_copy(degp3.at[1], rows.at[1])

      def dfill(r, carry):
        for jj in range(d // 16):
          slc = pl.ds(jj * 16, 16)
          deg = rows[0, r, slc] + rows[1, r, slc]
          x = jnp.maximum(deg, 1e-12)
          yi = 0x5F3759DF - (plsc.bitcast(x, jnp.int32) >> 1)
          y = plsc.bitcast(yi, jnp.float32)
          hx = 0.5 * x
          y = y * (1.5 - hx * y * y)
          y = y * (1.5 - hx * y * y)
          y = y * (1.5 - hx * y * y)
          y = jnp.where(deg > 0.0, y, 0.0)
          dinv_v[pl.ds(r * d + jj * 16, 16)] = y
        return carry

      lax.fori_loop(0, rows.shape[1], dfill, 0)

    def zfill(r, carry):
      for jj in range(d // 16):
        rows[0, r, pl.ds(jj * 16, 16)] = jnp.zeros((16,), jnp.float32)
      return carry

    lax.fori_loop(0, C, zfill, 0)
    for k in range(rows_per // C):
      pltpu.sync_copy(rows.at[0], acc.at[pl.ds(sid * rows_per + k * C, C)])
    plsc.subcore_barrier()

    def idx_start(cn, slot):
      pltpu.async_copy(src3.at[t, cn], src_c.at[slot], i_sem.at[slot])
      pltpu.async_copy(dst3.at[t, cn], dst_c.at[slot], i_sem.at[slot])
      pltpu.async_copy(aux3.at[t, cn], aux_c.at[slot], i_sem.at[slot])

    def idx_wait(cn, slot):
      pltpu.make_async_copy(src3.at[t, cn], src_c.at[slot],
                            i_sem.at[slot]).wait()
      pltpu.make_async_copy(dst3.at[t, cn], dst_c.at[slot],
                            i_sem.at[slot]).wait()
      pltpu.make_async_copy(aux3.at[t, cn], aux_c.at[slot],
                            i_sem.at[slot]).wait()

    grows = rows_bf if packed else rows

    def gather_start(slot, p):
      pltpu.async_copy(xw.at[src_c.at[slot]], grows.at[p], g_sem.at[p])

    def gather_wait(p):
      pltpu.make_async_copy(xw.at[pl.ds(0, C)], grows.at[p],
                            g_sem.at[p]).wait()

    def scatter_start(slot, p):
      pltpu.async_copy(rows.at[p], acc.at[dst_c.at[slot]], s_sem.at[p],
                       priority=1, add=True)

    def scatter_wait(p):
      pltpu.make_async_copy(rows.at[p], acc.at[pl.ds(0, C)],
                            s_sem.at[p]).wait()

    pltpu.sync_copy(src3.at[t, 0], src_c.at[0])
    pltpu.sync_copy(dst3.at[t, 0], dst_c.at[0])
    pltpu.sync_copy(aux3.at[t, 0], aux_c.at[0])
    gather_start(0, 0)
    idx_start(1, 1)

    def quad(kk, carry):
      for j in range(4):
        ci = kk * 4 + j
        p = j % 2
        q = (j + 1) % 2
        gather_wait(p)
        if not packed:
          if j == 0:
            @pl.when(kk > 0)
            def _():
              scatter_wait(q)
          else:
            scatter_wait(q)
        def issue_next():
          idx_wait(ci + 1, (j + 1) % 4)
          gather_start((j + 1) % 4, q)

        def issue_idx2():
          idx_start(ci + 2, (j + 2) % 4)

        if packed:
          if j < 3:
            issue_next()
          else:
            @pl.when(kk < K - 1)
            def _():
              issue_next()
          if j < 2:
            @pl.when(kk > 0)
            def _():
              scatter_wait(p)
          else:
            scatter_wait(p)
          if j < 2:
            issue_idx2()
          else:
            @pl.when(kk < K - 1)
            def _():
              issue_idx2()
        else:
          if j < 3:
            issue_next()
            if j < 2:
              issue_idx2()
            else:
              @pl.when(kk < K - 1)
              def _():
                issue_idx2()
          else:
            @pl.when(kk < K - 1)
            def _():
              issue_next()
              issue_idx2()

        if compute_norm:
          if j < 2:
            @pl.when(kk > 0)
            def _():
              pltpu.make_async_copy(normb.at[p], norm3.at[t, ci],
                                    n_sem.at[p]).wait()
          else:
            pltpu.make_async_copy(normb.at[p], norm3.at[t, ci],
                                  n_sem.at[p]).wait()

          def ngroup(g, c2):
            s16 = src_c[j, pl.ds(g * 16, 16)]
            d16 = dst_c[j, pl.ds(g * 16, 16)]
            e16 = aux_c[j, pl.ds(g * 16, 16)]
            normb[p, pl.ds(g * 16, 16)] = (
                plsc.load_gather(dinv_v, [s16]) * e16 *
                plsc.load_gather(dinv_v, [d16]))
            return c2

          lax.fori_loop(0, C // 16, ngroup, 0)
          pltpu.async_copy(normb.at[p], norm3.at[t, ci], n_sem.at[p])

        if packed:
          iota16 = lax.iota(jnp.int32, 16)

          @plsc.parallel_loop(0, C, unroll=4)
          def scale(ee):
            nsp = plsc.load_gather(
                aux_c, [jnp.full((16,), j, jnp.int32),
                        jnp.full((16,), ee, jnp.int32)])
            fp = jnp.full((16,), p, jnp.int32)
            fe = jnp.full((16,), ee, jnp.int32)
            for jj in range(d // 32):
              w32 = rows_bf[p, ee, pl.ds(jj * 32, 32)]
              w16 = plsc.bitcast(w32, jnp.int32)
              ev = plsc.bitcast(w16 << 16, jnp.float32) * nsp
              od = plsc.bitcast(w16 & jnp.int32(-65536), jnp.float32) * nsp
              plsc.store_scatter(
                  rows, [fp, fe, iota16 * 2 + (jj * 32)], ev)
              plsc.store_scatter(
                  rows, [fp, fe, iota16 * 2 + (jj * 32 + 1)], od)
        else:

          @plsc.parallel_loop(0, C, unroll=8)
          def scale(ee):
            if compute_norm:
              nsp = plsc.load_gather(
                  normb, [jnp.full((16,), p, jnp.int32),
                          jnp.full((16,), ee, jnp.int32)])
            else:
              nsp = plsc.load_gather(
                  aux_c, [jnp.full((16,), j, jnp.int32),
                          jnp.full((16,), ee, jnp.int32)])
            for jj in range(d // 16):
              slc = pl.ds(jj * 16, 16)
              rows[p, ee, slc] = rows[p, ee, slc] * nsp

        scatter_start(j, p)
      return carry

    lax.fori_loop(0, K, quad, 0)
    if packed:
      scatter_wait(0)
    scatter_wait(1)
    if compute_norm:
      pltpu.make_async_copy(normb.at[0], norm3.at[t, nchunk - 2],
                            n_sem.at[0]).wait()
      pltpu.make_async_copy(normb.at[1], norm3.at[t, nchunk - 1],
                            n_sem.at[1]).wait()
    plsc.subcore_barrier()
    pltpu.sync_copy(acc.at[pl.ds(sid * rows_per, rows_per)],
                    part.at[cid, pl.ds(sid * rows_per, rows_per)])

  return body


def kernel(x, edge_index, edge_weight, train_mask, y, W1, b1, W2, b2):
  n, d = x.shape
  e = edge_weight.shape[0]
  npad = ((n + 16 * 128 - 1) // (16 * 128)) * (16 * 128)
  ef = e + n
  epad = ((ef + NW * C * 4 - 1) // (NW * C * 4)) * (NW * C * 4)
  nchunk = epad // (NW * C)
  f32 = jnp.float32

  loop = jnp.arange(n, dtype=jnp.int32)
  padn = epad - ef
  padidx = jnp.arange(padn, dtype=jnp.int32) % n
  src3 = jnp.concatenate([edge_index[0], loop, padidx]).reshape(NW, nchunk, C)
  dst3 = jnp.concatenate([edge_index[1], loop, padidx]).reshape(NW, nchunk, C)
  ew3 = jnp.concatenate(
      [edge_weight, jnp.ones((n,), f32), jnp.zeros((padn,), f32)]
  ).reshape(NW, nchunk, C)
  xpad = jnp.pad(x, ((0, npad - n), (0, 0)))

  mesh = plsc.VectorSubcoreMesh(
      core_axis_name="c", subcore_axis_name="s", num_cores=NC,
      num_subcores=NS)

  degp = pl.kernel(
      functools.partial(_deg_body, nchunk, npad),
      out_type=jax.ShapeDtypeStruct((NC, npad), f32),
      mesh=mesh,
      compiler_params=pltpu.CompilerParams(needs_layout_passes=False),
      scratch_types=[
          pltpu.VMEM((nchunk, C), jnp.int32),
          pltpu.VMEM((nchunk, C), f32),
          pltpu.VMEM((npad // NS,), f32),
          pltpu.SemaphoreType.DMA((2,)),
          pltpu.VMEM_SHARED((npad,), f32),
      ],
  )(dst3, ew3)
  degp3 = degp.reshape(NC, npad // 128, 128)

  nb = npad // 1024

  def mm1_body(x_ref, w_ref, xw_ref):
    xw_ref[...] = jnp.dot(x_ref[...], w_ref[...],
                          preferred_element_type=f32)

  xw1 = pl.pallas_call(
      mm1_body,
      grid=(nb,),
      in_specs=[
          pl.BlockSpec((1024, d), lambda i: (i, 0)),
          pl.BlockSpec((d, d), lambda i: (0, 0)),
      ],
      out_specs=pl.BlockSpec((1024, d), lambda i: (i, 0)),
      out_shape=jax.ShapeDtypeStruct((npad, d), f32),
  )(xpad, W1)

  agg1 = pl.kernel(
      _make_agg(nchunk, n, npad, True),
      out_type=[
          jax.ShapeDtypeStruct((NW, nchunk, C), f32),
          jax.ShapeDtypeStruct((NC, npad, d), f32),
      ],
      mesh=mesh,
      compiler_params=pltpu.CompilerParams(needs_layout_passes=False),
      scratch_types=[
          pltpu.VMEM((4, C), jnp.int32),
          pltpu.VMEM((4, C), jnp.int32),
          pltpu.VMEM((4, C), f32),
          pltpu.VMEM((npad,), f32),
          pltpu.VMEM((2, C, d), f32),
          pltpu.VMEM((2, C), f32),
          pltpu.SemaphoreType.DMA((2,)),
          pltpu.SemaphoreType.DMA((2,)),
          pltpu.SemaphoreType.DMA((4,)),
          pltpu.SemaphoreType.DMA((2,)),
          pltpu.VMEM_SHARED((npad, d), f32),
      ],
  )
  norm3, part1 = agg1(src3, dst3, ew3, degp3, xw1)

  gb = 512
  ng = npad // gb

  def mm2_body(p_ref, b_ref, w_ref, out_ref):
    h = jnp.maximum(p_ref[0] + p_ref[1] + b_ref[...], 0.0)
    out_ref[...] = jnp.dot(h, w_ref[...], preferred_element_type=f32)

  xw2 = pl.pallas_call(
      mm2_body,
      grid=(ng,),
      in_specs=[
          pl.BlockSpec((NC, gb, d), lambda i: (0, i, 0)),
          pl.BlockSpec((1, d), lambda i: (0, 0)),
          pl.BlockSpec((d, d), lambda i: (0, 0)),
      ],
      out_specs=pl.BlockSpec((gb, d), lambda i: (i, 0)),
      out_shape=jax.ShapeDtypeStruct((npad, d), f32),
  )(part1, b1.reshape(1, d), W2)
  agg2 = pl.kernel(
      _make_agg(nchunk, n, npad, False),
      out_type=jax.ShapeDtypeStruct((NC, npad, d), f32),
      mesh=mesh,
      compiler_params=pltpu.CompilerParams(needs_layout_passes=False),
      scratch_types=[
          pltpu.VMEM((4, C), jnp.int32),
          pltpu.VMEM((4, C), jnp.int32),
          pltpu.VMEM((4, C), f32),
          pltpu.VMEM((2, C, d), f32),
          pltpu.SemaphoreType.DMA((2,)),
          pltpu.SemaphoreType.DMA((2,)),
          pltpu.SemaphoreType.DMA((4,)),
          pltpu.VMEM_SHARED((npad, d), f32),
      ],
  )
  part2 = agg2(src3, dst3, norm3, xw2)

  maskp = jnp.pad(train_mask, (0, npad - n))
  maskb = jnp.broadcast_to(maskp[:, None], (npad, d))

  def out_body(p_ref, b_ref, m_ref, out_ref):
    h = p_ref[0] + p_ref[1] + b_ref[...]
    out_ref[...] = jnp.where(m_ref[...], h, 0.0)

  h_sel = pl.pallas_call(
      out_body,
      grid=(ng,),
      in_specs=[
          pl.BlockSpec((NC, gb, d), lambda i: (0, i, 0)),
          pl.BlockSpec((1, d), lambda i: (0, 0)),
          pl.BlockSpec((gb, d), lambda i: (i, 0)),
      ],
      out_specs=pl.BlockSpec((gb, d), lambda i: (i, 0)),
      out_shape=jax.ShapeDtypeStruct((npad, d), f32),
  )(part2, b2.reshape(1, d), maskb)

  y_sel = jnp.where(train_mask, y, jnp.zeros((), dtype=y.dtype))
  return (h_sel[:n], y_sel)
